# baseline (speedup 1.0000x reference)
import jax
import jax.numpy as jnp
from jax import lax
from jax.experimental import pallas as pl
from jax.experimental.pallas import tpu as pltpu
from jax.experimental.pallas import tpu_sc as plsc

N_USERS = 10000
N_ITEMS = 10000
N_EDGES = 160000
D = 256
DH = 128

NC, NS = 2, 16


_K1_CH = 80
_K1_EPT = N_EDGES // NS
_K1_CPT = _K1_EPT // _K1_CH
_K1_RPT = 632
_K1_ACC = NS * _K1_RPT


def _k1_body(ulo, uhi, ilo, ihi, eu1, ei1, eu2, ei2, zinit, ones_hbm,
             out_item, out_user, out_cnt,
             acc, idxs1, idxd2, rows0, rows1, gsem, ssem0, ssem1):
    cid = lax.axis_index("c")
    sid = lax.axis_index("s")
    zbase = sid * _K1_RPT

    def drain_add(rows_b, ssem_b):
        pltpu.make_async_copy(rows_b, acc.at[idxd2.at[0]], ssem_b).wait()

    def run_pass(tabs, dst0, dst1, out):
        pltpu.sync_copy(zinit.at[pl.ds(zbase, _K1_RPT)],
                        acc.at[pl.ds(zbase, _K1_RPT)])
        if tabs is None:
            pltpu.sync_copy(ones_hbm, rows0)
        else:
            pltpu.sync_copy(tabs[2].at[pl.ds(sid * _K1_EPT, _K1_EPT)], idxs1)

        @pl.when(cid == 0)
        def _():
            pltpu.sync_copy(dst0.at[sid], idxd2)

        @pl.when(cid == 1)
        def _():
            pltpu.sync_copy(dst1.at[sid], idxd2)

        plsc.subcore_barrier()

        if tabs is None:
            def chunk(k, c):
                @pl.when(k >= 2)
                def _():
                    drain_add(rows0, ssem0)

                pltpu.async_copy(rows0, acc.at[idxd2.at[k]], ssem0, add=True)
                return c

            lax.fori_loop(0, _K1_CPT, chunk, 0)
            drain_add(rows0, ssem0)
            drain_add(rows0, ssem0)
        else:
            tab_lo, tab_hi = tabs[0], tabs[1]

            def fchunk(k2, b, rows_b, ssem_b):
                k = 2 * k2 + b

                @pl.when(k2 > 0)
                def _():
                    drain_add(rows_b, ssem_b)

                isl = idxs1.at[pl.ds(k * _K1_CH, _K1_CH)]

                @pl.when(cid == 0)
                def _():
                    pltpu.async_copy(tab_lo.at[isl], rows_b, gsem).wait()

                @pl.when(cid == 1)
                def _():
                    pltpu.async_copy(tab_hi.at[isl], rows_b, gsem).wait()

                pltpu.async_copy(rows_b, acc.at[idxd2.at[k]], ssem_b,
                                 add=True)

            def pair(k2, c):
                fchunk(k2, 0, rows0, ssem0)
                fchunk(k2, 1, rows1, ssem1)
                return c

            lax.fori_loop(0, _K1_CPT // 2, pair, 0)
            fchunk(_K1_CPT // 2, 0, rows0, ssem0)
            drain_add(rows0, ssem0)
            drain_add(rows1, ssem1)

        plsc.subcore_barrier()
        pltpu.sync_copy(acc.at[pl.ds(zbase, _K1_RPT)],
                        out.at[cid, pl.ds(zbase, _K1_RPT)])
        plsc.subcore_barrier()

    run_pass((ulo, uhi, eu1), ei2, ei2, out_item)
    run_pass((ilo, ihi, ei1), eu2, eu2, out_user)
    run_pass(None, ei2, eu2, out_cnt)


def _k1_call(ulo, uhi, ilo, ihi, eu1, ei1, eu2, ei2, zinit, ones_hbm):
    mesh = plsc.VectorSubcoreMesh(core_axis_name="c", subcore_axis_name="s",
                                  num_cores=NC, num_subcores=NS)
    f = pl.kernel(
        _k1_body,
        out_type=(
            jax.ShapeDtypeStruct((NC, _K1_ACC, DH), jnp.float32),
            jax.ShapeDtypeStruct((NC, _K1_ACC, DH), jnp.float32),
            jax.ShapeDtypeStruct((NC, _K1_ACC, DH), jnp.float32),
        ),
        mesh=mesh,
        scratch_types=[
            pltpu.VMEM_SHARED((_K1_ACC, DH), jnp.float32),
            pltpu.VMEM((_K1_EPT,), jnp.int32),
            pltpu.VMEM((_K1_CPT, _K1_CH), jnp.int32),
            pltpu.VMEM((_K1_CH, DH), jnp.float32),
            pltpu.VMEM((_K1_CH, DH), jnp.float32),
            pltpu.SemaphoreType.DMA,
            pltpu.SemaphoreType.DMA,
            pltpu.SemaphoreType.DMA,
        ],
    )
    return f(ulo, uhi, ilo, ihi, eu1, ei1, eu2, ei2, zinit, ones_hbm)


_K2_R = 2000


def _k2_body(silo, sihi, sulo, suhi, cnti, cntu, wu, bu, wi, bi,
             pu_ref, pi_ref):
    cnt_i = jnp.clip(cnti[0][:, 0:1], 1.0, None)
    h_i = jnp.concatenate([silo[0], sihi[0]], axis=1) / cnt_i
    p_i = jnp.dot(h_i, wi[...], preferred_element_type=jnp.float32) + bi[...]
    pi_ref[...] = p_i.astype(jnp.bfloat16)

    cnt_u = jnp.clip(cntu[0][:, 0:1], 1.0, None)
    h_u = jnp.concatenate([sulo[0], suhi[0]], axis=1) / cnt_u
    p_u = jnp.dot(h_u, wu[...], preferred_element_type=jnp.float32) + bu[...]
    pu_ref[...] = p_u.astype(jnp.bfloat16)


def _k2_call(sum_item, sum_user, cnt, fc_user_W, fc_user_b, fc_item_W,
             fc_item_b):
    R = _K2_R
    grid = (N_ITEMS // R,)
    return pl.pallas_call(
        _k2_body,
        grid=grid,
        in_specs=[
            pl.BlockSpec((1, R, DH), lambda i: (0, i, 0)),
            pl.BlockSpec((1, R, DH), lambda i: (1, i, 0)),
            pl.BlockSpec((1, R, DH), lambda i: (0, i, 0)),
            pl.BlockSpec((1, R, DH), lambda i: (1, i, 0)),
            pl.BlockSpec((1, R, DH), lambda i: (0, i, 0)),
            pl.BlockSpec((1, R, DH), lambda i: (1, i, 0)),
            pl.BlockSpec((D, D), lambda i: (0, 0)),
            pl.BlockSpec((1, D), lambda i: (0, 0)),
            pl.BlockSpec((D, D), lambda i: (0, 0)),
            pl.BlockSpec((1, D), lambda i: (0, 0)),
        ],
        out_specs=[
            pl.BlockSpec((R, D), lambda i: (i, 0)),
            pl.BlockSpec((R, D), lambda i: (i, 0)),
        ],
        out_shape=[
            jax.ShapeDtypeStruct((N_USERS, D), jnp.bfloat16),
            jax.ShapeDtypeStruct((N_ITEMS, D), jnp.bfloat16),
        ],
    )(sum_item, sum_item, sum_user, sum_user, cnt, cnt, fc_user_W,
      fc_user_b.reshape(1, D), fc_item_W, fc_item_b.reshape(1, D))


_K3_EPT = N_EDGES // (NC * NS)
_K3_CH = 40
_K3_CPT = _K3_EPT // _K3_CH


def _k3_body(uf, pu, itf, pi, eu2, ei2, bias_flat,
             p_all, m_all, q_all, n_all, bias_all,
             idxu2, idxi2,
             rp0, rm0, rq0, rn0, rb0, rp1, rm1, rq1, rn1, rb1,
             gsem, wsem0, wsem1):
    cid = lax.axis_index("c")
    sid = lax.axis_index("s")
    wid = sid * NC + cid
    tbase = wid * _K3_EPT

    pltpu.sync_copy(eu2.at[wid], idxu2)
    pltpu.sync_copy(ei2.at[wid], idxi2)

    def drain(bufs, wsem_b):
        rp, rm, rq, rn, rb = bufs
        base0 = pl.ds(tbase, _K3_CH)
        pltpu.make_async_copy(rp, p_all.at[base0], wsem_b).wait()
        pltpu.make_async_copy(rm, m_all.at[base0], wsem_b).wait()
        pltpu.make_async_copy(rq, q_all.at[base0], wsem_b).wait()
        pltpu.make_async_copy(rn, n_all.at[base0], wsem_b).wait()
        pltpu.make_async_copy(rb, bias_all.at[base0], wsem_b).wait()

    def fchunk(k2, b, bufs, wsem_b):
        k = 2 * k2 + b
        rp, rm, rq, rn, rb = bufs
        base = tbase + k * _K3_CH

        @pl.when(k2 > 0)
        def _():
            drain(bufs, wsem_b)

        iu = idxu2.at[k]
        ii = idxi2.at[k]
        c1 = pltpu.async_copy(uf.at[iu], rp, gsem)
        c2 = pltpu.async_copy(pu.at[iu], rm, gsem)
        c3 = pltpu.async_copy(itf.at[ii], rq, gsem)
        c4 = pltpu.async_copy(pi.at[ii], rn, gsem)
        c5 = pltpu.async_copy(bias_flat.at[ii], rb, gsem)
        c1.wait()
        c2.wait()
        c3.wait()
        c4.wait()
        c5.wait()
        sl = pl.ds(base, _K3_CH)
        pltpu.async_copy(rp, p_all.at[sl], wsem_b)
        pltpu.async_copy(rm, m_all.at[sl], wsem_b)
        pltpu.async_copy(rq, q_all.at[sl], wsem_b)
        pltpu.async_copy(rn, n_all.at[sl], wsem_b)
        pltpu.async_copy(rb, bias_all.at[sl], wsem_b)

    bufs0 = (rp0, rm0, rq0, rn0, rb0)
    bufs1 = (rp1, rm1, rq1, rn1, rb1)

    def pair(k2, c):
        fchunk(k2, 0, bufs0, wsem0)
        fchunk(k2, 1, bufs1, wsem1)
        return c

    lax.fori_loop(0, _K3_CPT // 2, pair, 0)
    fchunk(_K3_CPT // 2, 0, bufs0, wsem0)
    drain(bufs0, wsem0)
    drain(bufs1, wsem1)


def _k3_call(user_feat, prop_user, item_feat, prop_item, eu2, ei2, bias_flat):
    mesh = plsc.VectorSubcoreMesh(core_axis_name="c", subcore_axis_name="s",
                                  num_cores=NC, num_subcores=NS)
    rowbuf = pltpu.VMEM((_K3_CH, DH), jnp.int32)
    epk = jax.ShapeDtypeStruct((N_EDGES, DH), jnp.int32)
    f = pl.kernel(
        _k3_body,
        out_type=(
            epk, epk, epk, epk,
            jax.ShapeDtypeStruct((N_EDGES,), jnp.float32),
        ),
        mesh=mesh,
        scratch_types=[
            pltpu.VMEM((_K3_CPT, _K3_CH), jnp.int32),
            pltpu.VMEM((_K3_CPT, _K3_CH), jnp.int32),
            rowbuf, rowbuf, rowbuf, rowbuf,
            pltpu.VMEM((_K3_CH,), jnp.float32),
            rowbuf, rowbuf, rowbuf, rowbuf,
            pltpu.VMEM((_K3_CH,), jnp.float32),
            pltpu.SemaphoreType.DMA,
            pltpu.SemaphoreType.DMA,
            pltpu.SemaphoreType.DMA,
        ],
    )
    return f(user_feat, prop_user, item_feat, prop_item, eu2, ei2, bias_flat)


_K4_E = 1000


def _unpack_bf16(r):
    w = r[...]
    lo = jax.lax.bitcast_convert_type(jax.lax.shift_left(w, 16), jnp.float32)
    hi = jax.lax.bitcast_convert_type(w, jnp.float32)
    return lo, hi


def _k4_body(p_ref, m_ref, q_ref, n_ref, bias_ref, w1, b1, w2, b2, w3, b3,
             out_ref):
    pe, po = _unpack_bf16(p_ref)
    me, mo = _unpack_bf16(m_ref)
    qe, qo = _unpack_bf16(q_ref)
    ne, no = _unpack_bf16(n_ref)
    x = jnp.concatenate([pe * qe, po * qo, pe * me, po * mo,
                         ne * qe, no * qo, ne * me, no * mo],
                        axis=1).astype(jnp.bfloat16)
    y = jnp.dot(x, w1[...], preferred_element_type=jnp.float32) + b1[...]
    y = jnp.maximum(y, 0.0).astype(jnp.bfloat16)
    z = jnp.dot(y, w2[...], preferred_element_type=jnp.float32) + b2[...]
    z = jnp.maximum(z, 0.0)
    o = jnp.dot(z, w3[...], preferred_element_type=jnp.float32) + b3[...]
    out_ref[...] = o + bias_ref[...]


def _k4_call(p_all, m_all, q_all, n_all, bias_all, W1, b1, W2, b2, W3, b3):
    E = _K4_E
    grid = (N_EDGES // E,)
    return pl.pallas_call(
        _k4_body,
        grid=grid,
        in_specs=[
            pl.BlockSpec((E, DH), lambda i: (i, 0)),
            pl.BlockSpec((E, DH), lambda i: (i, 0)),
            pl.BlockSpec((E, DH), lambda i: (i, 0)),
            pl.BlockSpec((E, DH), lambda i: (i, 0)),
            pl.BlockSpec((E, 1), lambda i: (i, 0)),
            pl.BlockSpec((4 * D, D), lambda i: (0, 0)),
            pl.BlockSpec((1, D), lambda i: (0, 0)),
            pl.BlockSpec((D, 64), lambda i: (0, 0)),
            pl.BlockSpec((1, 64), lambda i: (0, 0)),
            pl.BlockSpec((64, 1), lambda i: (0, 0)),
            pl.BlockSpec((1, 1), lambda i: (0, 0)),
        ],
        out_specs=pl.BlockSpec((E, 1), lambda i: (i, 0)),
        out_shape=jax.ShapeDtypeStruct((N_EDGES, 1), jnp.float32),
    )(p_all, m_all, q_all, n_all, bias_all.reshape(N_EDGES, 1),
      _permute_w1(W1).astype(jnp.bfloat16), b1.reshape(1, D),
      W2.astype(jnp.bfloat16), b2.reshape(1, 64), W3, b3.reshape(1, 1))


def _permute_w1(W1):
    import numpy as np
    half = np.concatenate([np.arange(0, D, 2), np.arange(1, D, 2)])
    perm = np.concatenate([g * D + half for g in range(4)])
    return W1[perm]


def kernel(user_feat, item_feat, user_bias, item_bias, fc_user_W, fc_user_b,
           fc_item_W, fc_item_b, W1, b1, W2, b2, W3, b3, edge_users,
           edge_items):
    f32 = jnp.float32
    ulo = user_feat[:, :DH]
    uhi = user_feat[:, DH:]
    ilo = item_feat[:, :DH]
    ihi = item_feat[:, DH:]
    zinit = jnp.zeros((_K1_ACC, DH), f32)
    ones80 = jnp.ones((_K1_CH, DH), f32)
    eu2 = edge_users.reshape(NS, _K1_CPT, _K1_CH)
    ei2 = edge_items.reshape(NS, _K1_CPT, _K1_CH)
    eu2b = edge_users.reshape(NC * NS, _K3_CPT, _K3_CH)
    ei2b = edge_items.reshape(NC * NS, _K3_CPT, _K3_CH)
    bias_flat = item_bias.reshape(-1)

    bf16 = jnp.bfloat16

    def pack_bf16(a):
        b = a.astype(bf16).reshape(a.shape[0], DH, 2)
        return jax.lax.bitcast_convert_type(b, jnp.int32)

    ubf = pack_bf16(user_feat)
    ibf = pack_bf16(item_feat)

    sum_item, sum_user, cnt = _k1_call(ulo, uhi, ilo, ihi, edge_users,
                                       edge_items, eu2, ei2, zinit, ones80)
    prop_user, prop_item = _k2_call(sum_item, sum_user, cnt, fc_user_W,
                                    fc_user_b, fc_item_W, fc_item_b)
    p_pk, m_pk, q_pk, n_pk, bias_all = _k3_call(
        ubf, pack_bf16(prop_user), ibf, pack_bf16(prop_item),
        eu2b, ei2b, bias_flat)
    return _k4_call(p_pk, m_pk, q_pk, n_pk, bias_all, W1, b1, W2, b2,
                    W3, b3)


# speedup vs baseline: 1.4138x; 1.4138x over previous
import jax
import jax.numpy as jnp
import numpy as np
from jax import lax
from jax.experimental import pallas as pl
from jax.experimental.pallas import tpu as pltpu
from jax.experimental.pallas import tpu_sc as plsc

N_USERS = 10000
N_ITEMS = 10000
N_EDGES = 160000
D = 256
DH = 128

NC, NS = 2, 16


_K1_CH = 80
_K1_EPT = N_EDGES // NS
_K1_CPT = _K1_EPT // _K1_CH
_K1_RPT = 632
_K1_ACC = NS * _K1_RPT


def _k1_body(ulo, uhi, ilo, ihi, eu1, ei1, zinit, ones_hbm,
             out_item, out_user, out_cnt,
             acc, idxs1, idxd1, rows0, rows1, gsem0, gsem1, ssem0, ssem1):
    cid = lax.axis_index("c")
    sid = lax.axis_index("s")
    zbase = sid * _K1_RPT

    def dsl(k):
        return idxd1.at[pl.ds(k * _K1_CH, _K1_CH)]

    def drain_add(rows_b, ssem_b):
        pltpu.make_async_copy(rows_b, acc.at[dsl(0)], ssem_b).wait()

    def run_pass(tabs, dst0, dst1, out):
        pltpu.sync_copy(zinit.at[pl.ds(zbase, _K1_RPT)],
                        acc.at[pl.ds(zbase, _K1_RPT)])
        if tabs is None:
            pltpu.sync_copy(ones_hbm, rows0)
        else:
            pltpu.sync_copy(tabs[2].at[pl.ds(sid * _K1_EPT, _K1_EPT)], idxs1)

        @pl.when(cid == 0)
        def _():
            pltpu.sync_copy(dst0.at[pl.ds(sid * _K1_EPT, _K1_EPT)], idxd1)

        @pl.when(cid == 1)
        def _():
            pltpu.sync_copy(dst1.at[pl.ds(sid * _K1_EPT, _K1_EPT)], idxd1)

        plsc.subcore_barrier()

        if tabs is None:
            def chunk(k, c):
                @pl.when(k >= 2)
                def _():
                    drain_add(rows0, ssem0)

                pltpu.async_copy(rows0, acc.at[dsl(k)], ssem0, add=True)
                return c

            lax.fori_loop(0, _K1_CPT, chunk, 0)
            drain_add(rows0, ssem0)
            drain_add(rows0, ssem0)
        else:
            tab_lo, tab_hi = tabs[0], tabs[1]

            def issue_gather(k, rows_b, gsem_b):
                isl = idxs1.at[pl.ds(k * _K1_CH, _K1_CH)]

                @pl.when(cid == 0)
                def _():
                    pltpu.async_copy(tab_lo.at[isl], rows_b, gsem_b)

                @pl.when(cid == 1)
                def _():
                    pltpu.async_copy(tab_hi.at[isl], rows_b, gsem_b)

            def wait_gather(rows_b, gsem_b):
                pltpu.make_async_copy(
                    tab_lo.at[idxs1.at[pl.ds(0, _K1_CH)]], rows_b,
                    gsem_b).wait()

            issue_gather(0, rows0, gsem0)

            def fchunk(k, rows_b, gsem_b, ssem_b, rows_n, gsem_n, ssem_n):
                @pl.when(k + 1 < _K1_CPT)
                def _():
                    @pl.when(k >= 1)
                    def _():
                        drain_add(rows_n, ssem_n)

                    issue_gather(k + 1, rows_n, gsem_n)

                wait_gather(rows_b, gsem_b)
                pltpu.async_copy(rows_b, acc.at[dsl(k)], ssem_b, add=True)

            def pair(k2, c):
                fchunk(2 * k2, rows0, gsem0, ssem0, rows1, gsem1, ssem1)
                fchunk(2 * k2 + 1, rows1, gsem1, ssem1, rows0, gsem0, ssem0)
                return c

            lax.fori_loop(0, _K1_CPT // 2, pair, 0)
            fchunk(_K1_CPT - 1, rows0, gsem0, ssem0, rows1, gsem1, ssem1)
            drain_add(rows0, ssem0)
            drain_add(rows1, ssem1)

        plsc.subcore_barrier()
        pltpu.sync_copy(acc.at[pl.ds(zbase, _K1_RPT)],
                        out.at[cid, pl.ds(zbase, _K1_RPT)])
        plsc.subcore_barrier()

    run_pass((ulo, uhi, eu1), ei1, ei1, out_item)
    run_pass((ilo, ihi, ei1), eu1, eu1, out_user)
    run_pass(None, ei1, eu1, out_cnt)


def _k1_call(ulo, uhi, ilo, ihi, eu1, ei1, zinit, ones_hbm):
    mesh = plsc.VectorSubcoreMesh(core_axis_name="c", subcore_axis_name="s",
                                  num_cores=NC, num_subcores=NS)
    f = pl.kernel(
        _k1_body,
        out_type=(
            jax.ShapeDtypeStruct((NC, _K1_ACC, DH), jnp.float32),
            jax.ShapeDtypeStruct((NC, _K1_ACC, DH), jnp.float32),
            jax.ShapeDtypeStruct((NC, _K1_ACC, DH), jnp.float32),
        ),
        mesh=mesh,
        scratch_types=[
            pltpu.VMEM_SHARED((_K1_ACC, DH), jnp.float32),
            pltpu.VMEM((_K1_EPT,), jnp.int32),
            pltpu.VMEM((_K1_EPT,), jnp.int32),
            pltpu.VMEM((_K1_CH, DH), jnp.float32),
            pltpu.VMEM((_K1_CH, DH), jnp.float32),
            pltpu.SemaphoreType.DMA,
            pltpu.SemaphoreType.DMA,
            pltpu.SemaphoreType.DMA,
            pltpu.SemaphoreType.DMA,
        ],
    )
    return f(ulo, uhi, ilo, ihi, eu1, ei1, zinit, ones_hbm)


_K2_R = 2000

_PERM = np.concatenate([np.arange(0, D, 2), np.arange(1, D, 2)])


def _pack_rtne(x):
    b = lax.bitcast_convert_type(x, jnp.int32)
    r = lax.shift_right_logical(
        b + 0x7FFF + (lax.shift_right_logical(b, 16) & 1), 16) & 0xFFFF
    return r[:, :DH] | lax.shift_left(r[:, DH:], 16)


def _k2_body(silo, sihi, sulo, suhi, cnti, cntu, wu, bu, wi, bi,
             pu_ref, pi_ref):
    cnt_i = jnp.clip(cnti[0][:, 0:1], 1.0, None)
    h_i = jnp.concatenate([silo[0], sihi[0]], axis=1) / cnt_i
    p_i = jnp.dot(h_i, wi[...], preferred_element_type=jnp.float32) + bi[...]
    pi_ref[...] = _pack_rtne(p_i)

    cnt_u = jnp.clip(cntu[0][:, 0:1], 1.0, None)
    h_u = jnp.concatenate([sulo[0], suhi[0]], axis=1) / cnt_u
    p_u = jnp.dot(h_u, wu[...], preferred_element_type=jnp.float32) + bu[...]
    pu_ref[...] = _pack_rtne(p_u)


def _k2_call(sum_item, sum_user, cnt, fc_user_W, fc_user_b, fc_item_W,
             fc_item_b):
    R = _K2_R
    grid = (N_ITEMS // R,)
    return pl.pallas_call(
        _k2_body,
        grid=grid,
        in_specs=[
            pl.BlockSpec((1, R, DH), lambda i: (0, i, 0)),
            pl.BlockSpec((1, R, DH), lambda i: (1, i, 0)),
            pl.BlockSpec((1, R, DH), lambda i: (0, i, 0)),
            pl.BlockSpec((1, R, DH), lambda i: (1, i, 0)),
            pl.BlockSpec((1, R, DH), lambda i: (0, i, 0)),
            pl.BlockSpec((1, R, DH), lambda i: (1, i, 0)),
            pl.BlockSpec((D, D), lambda i: (0, 0)),
            pl.BlockSpec((1, D), lambda i: (0, 0)),
            pl.BlockSpec((D, D), lambda i: (0, 0)),
            pl.BlockSpec((1, D), lambda i: (0, 0)),
        ],
        out_specs=[
            pl.BlockSpec((R, DH), lambda i: (i, 0)),
            pl.BlockSpec((R, DH), lambda i: (i, 0)),
        ],
        out_shape=[
            jax.ShapeDtypeStruct((N_USERS, DH), jnp.int32),
            jax.ShapeDtypeStruct((N_ITEMS, DH), jnp.int32),
        ],
    )(sum_item, sum_item, sum_user, sum_user, cnt, cnt, fc_user_W[:, _PERM],
      fc_user_b[_PERM].reshape(1, D), fc_item_W[:, _PERM],
      fc_item_b[_PERM].reshape(1, D))


_K3_EPT = N_EDGES // (NC * NS)
_K3_CH = 40
_K3_CPT = _K3_EPT // _K3_CH


def _k3_body(uf, pu, itf, pi, eu, ei, bias_flat,
             p_all, m_all, q_all, n_all, bias_all,
             idxu1, idxi1,
             rp0, rm0, rq0, rn0, rb0, rp1, rm1, rq1, rn1, rb1,
             gsem0, gsem1, wsem0, wsem1):
    cid = lax.axis_index("c")
    sid = lax.axis_index("s")
    wid = sid * NC + cid
    tbase = wid * _K3_EPT

    pltpu.sync_copy(eu.at[pl.ds(tbase, _K3_EPT)], idxu1)
    pltpu.sync_copy(ei.at[pl.ds(tbase, _K3_EPT)], idxi1)

    def drain_writes(bufs, wsem_b):
        rp, rm, rq, rn, rb = bufs
        base0 = pl.ds(tbase, _K3_CH)
        pltpu.make_async_copy(rp, p_all.at[base0], wsem_b).wait()
        pltpu.make_async_copy(rm, m_all.at[base0], wsem_b).wait()
        pltpu.make_async_copy(rq, q_all.at[base0], wsem_b).wait()
        pltpu.make_async_copy(rn, n_all.at[base0], wsem_b).wait()
        pltpu.make_async_copy(rb, bias_all.at[base0], wsem_b).wait()

    def issue_gathers(k, bufs, gsem_b):
        rp, rm, rq, rn, rb = bufs
        iu = idxu1.at[pl.ds(k * _K3_CH, _K3_CH)]
        ii = idxi1.at[pl.ds(k * _K3_CH, _K3_CH)]
        pltpu.async_copy(uf.at[iu], rp, gsem_b)
        pltpu.async_copy(pu.at[iu], rm, gsem_b)
        pltpu.async_copy(itf.at[ii], rq, gsem_b)
        pltpu.async_copy(pi.at[ii], rn, gsem_b)
        pltpu.async_copy(bias_flat.at[ii], rb, gsem_b)

    def wait_gathers(bufs, gsem_b):
        rp, rm, rq, rn, rb = bufs
        iu0 = idxu1.at[pl.ds(0, _K3_CH)]
        pltpu.make_async_copy(uf.at[iu0], rp, gsem_b).wait()
        pltpu.make_async_copy(pu.at[iu0], rm, gsem_b).wait()
        pltpu.make_async_copy(itf.at[iu0], rq, gsem_b).wait()
        pltpu.make_async_copy(pi.at[iu0], rn, gsem_b).wait()
        pltpu.make_async_copy(bias_flat.at[iu0], rb, gsem_b).wait()

    bufs0 = (rp0, rm0, rq0, rn0, rb0)
    bufs1 = (rp1, rm1, rq1, rn1, rb1)

    issue_gathers(0, bufs0, gsem0)

    def fchunk(k, bufs_b, gsem_b, wsem_b, bufs_n, gsem_n, wsem_n):
        @pl.when(k + 1 < _K3_CPT)
        def _():
            @pl.when(k >= 1)
            def _():
                drain_writes(bufs_n, wsem_n)

            issue_gathers(k + 1, bufs_n, gsem_n)

        wait_gathers(bufs_b, gsem_b)
        rp, rm, rq, rn, rb = bufs_b
        sl = pl.ds(tbase + k * _K3_CH, _K3_CH)
        pltpu.async_copy(rp, p_all.at[sl], wsem_b)
        pltpu.async_copy(rm, m_all.at[sl], wsem_b)
        pltpu.async_copy(rq, q_all.at[sl], wsem_b)
        pltpu.async_copy(rn, n_all.at[sl], wsem_b)
        pltpu.async_copy(rb, bias_all.at[sl], wsem_b)

    def pair(k2, c):
        fchunk(2 * k2, bufs0, gsem0, wsem0, bufs1, gsem1, wsem1)
        fchunk(2 * k2 + 1, bufs1, gsem1, wsem1, bufs0, gsem0, wsem0)
        return c

    lax.fori_loop(0, _K3_CPT // 2, pair, 0)
    fchunk(_K3_CPT - 1, bufs0, gsem0, wsem0, bufs1, gsem1, wsem1)
    drain_writes(bufs0, wsem0)
    drain_writes(bufs1, wsem1)


def _k3_call(user_feat, prop_user, item_feat, prop_item, eu, ei, bias_flat):
    mesh = plsc.VectorSubcoreMesh(core_axis_name="c", subcore_axis_name="s",
                                  num_cores=NC, num_subcores=NS)
    rowbuf = pltpu.VMEM((_K3_CH, DH), jnp.int32)
    epk = jax.ShapeDtypeStruct((N_EDGES, DH), jnp.int32)
    f = pl.kernel(
        _k3_body,
        out_type=(
            epk, epk, epk, epk,
            jax.ShapeDtypeStruct((N_EDGES,), jnp.float32),
        ),
        mesh=mesh,
        scratch_types=[
            pltpu.VMEM((_K3_EPT,), jnp.int32),
            pltpu.VMEM((_K3_EPT,), jnp.int32),
            rowbuf, rowbuf, rowbuf, rowbuf,
            pltpu.VMEM((_K3_CH,), jnp.float32),
            rowbuf, rowbuf, rowbuf, rowbuf,
            pltpu.VMEM((_K3_CH,), jnp.float32),
            pltpu.SemaphoreType.DMA,
            pltpu.SemaphoreType.DMA,
            pltpu.SemaphoreType.DMA,
            pltpu.SemaphoreType.DMA,
        ],
    )
    return f(user_feat, prop_user, item_feat, prop_item, eu, ei, bias_flat)


_K4_E = 2000


def _unpack_bf16(r):
    bf = jnp.bfloat16
    w = r[...]
    lo = lax.bitcast_convert_type(lax.shift_left(w, 16), jnp.float32)
    hi = lax.bitcast_convert_type(w, jnp.float32)
    return lo.astype(bf), hi.astype(bf)


def _k4_body(p_ref, m_ref, q_ref, n_ref, bias_ref, w1, b1, w2, b2, w3, b3,
             out_ref):
    pe, po = _unpack_bf16(p_ref)
    me, mo = _unpack_bf16(m_ref)
    qe, qo = _unpack_bf16(q_ref)
    ne, no = _unpack_bf16(n_ref)
    x = jnp.concatenate([pe * qe, po * qo, pe * me, po * mo,
                         ne * qe, no * qo, ne * me, no * mo], axis=1)
    y = jnp.dot(x, w1[...], preferred_element_type=jnp.float32) + b1[...]
    y = jnp.maximum(y, 0.0).astype(jnp.bfloat16)
    z = jnp.dot(y, w2[...], preferred_element_type=jnp.float32) + b2[...]
    z = jnp.maximum(z, 0.0)
    o = jnp.dot(z, w3[...], preferred_element_type=jnp.float32) + b3[...]
    out_ref[...] = o + bias_ref[...]


def _k4_call(p_all, m_all, q_all, n_all, bias_all, W1, b1, W2, b2, W3, b3):
    E = _K4_E
    grid = (N_EDGES // E,)
    return pl.pallas_call(
        _k4_body,
        grid=grid,
        in_specs=[
            pl.BlockSpec((E, DH), lambda i: (i, 0)),
            pl.BlockSpec((E, DH), lambda i: (i, 0)),
            pl.BlockSpec((E, DH), lambda i: (i, 0)),
            pl.BlockSpec((E, DH), lambda i: (i, 0)),
            pl.BlockSpec((E, 1), lambda i: (i, 0)),
            pl.BlockSpec((4 * D, D), lambda i: (0, 0)),
            pl.BlockSpec((1, D), lambda i: (0, 0)),
            pl.BlockSpec((D, 64), lambda i: (0, 0)),
            pl.BlockSpec((1, 64), lambda i: (0, 0)),
            pl.BlockSpec((64, 1), lambda i: (0, 0)),
            pl.BlockSpec((1, 1), lambda i: (0, 0)),
        ],
        out_specs=pl.BlockSpec((E, 1), lambda i: (i, 0)),
        out_shape=jax.ShapeDtypeStruct((N_EDGES, 1), jnp.float32),
    )(p_all, m_all, q_all, n_all, bias_all.reshape(N_EDGES, 1),
      _permute_w1(W1).astype(jnp.bfloat16), b1.reshape(1, D),
      W2.astype(jnp.bfloat16), b2.reshape(1, 64), W3, b3.reshape(1, 1))


def _permute_w1(W1):
    perm = np.concatenate([g * D + _PERM for g in range(4)])
    return W1[perm]


def kernel(user_feat, item_feat, user_bias, item_bias, fc_user_W, fc_user_b,
           fc_item_W, fc_item_b, W1, b1, W2, b2, W3, b3, edge_users,
           edge_items):
    f32 = jnp.float32
    ulo = user_feat[:, :DH]
    uhi = user_feat[:, DH:]
    ilo = item_feat[:, :DH]
    ihi = item_feat[:, DH:]
    zinit = jnp.zeros((_K1_ACC, DH), f32)
    ones80 = jnp.ones((_K1_CH, DH), f32)
    bias_flat = item_bias.reshape(-1)

    bf16 = jnp.bfloat16

    def pack_bf16(a):
        b = a.astype(bf16).reshape(a.shape[0], DH, 2)
        return jax.lax.bitcast_convert_type(b, jnp.int32)

    ubf = pack_bf16(user_feat)
    ibf = pack_bf16(item_feat)

    sum_item, sum_user, cnt = _k1_call(ulo, uhi, ilo, ihi, edge_users,
                                       edge_items, zinit, ones80)
    prop_user, prop_item = _k2_call(sum_item, sum_user, cnt, fc_user_W,
                                    fc_user_b, fc_item_W, fc_item_b)
    p_pk, m_pk, q_pk, n_pk, bias_all = _k3_call(
        ubf, prop_user, ibf, prop_item, edge_users, edge_items, bias_flat)
    return _k4_call(p_pk, m_pk, q_pk, n_pk, bias_all, W1, b1, W2, b2,
                    W3, b3)
